# baseline (speedup 1.0000x reference)
import jax
import jax.numpy as jnp
from jax.experimental import pallas as pl

_SEQ_BLK = 512
_LG_BLK = 1024


def _emb_body(binT_ref, emb_ref, out_ref):
    ampT = binT_ref[...] * 2.0 - 1.0
    table = emb_ref[...]
    out_ref[...] = ampT.T[:, :, None] * table[None, :, :]


def _logit_body(binT_ref, emb_ref, out_ref):
    ampT = binT_ref[...] * 2.0 - 1.0
    rowsum = jnp.sum(emb_ref[...], axis=1)
    scaled = ampT * rowsum[:, None]
    for k in range(_LG_BLK // 128):
        out_ref[:, k, :] = scaled[:, k * 128:(k + 1) * 128]


def kernel(binary_input, embeddings):
    seq_len, blen = binary_input.shape
    vocab, emb_sz = embeddings.shape
    binT = binary_input.T

    emb = pl.pallas_call(
        _emb_body,
        grid=(seq_len // _SEQ_BLK,),
        in_specs=[
            pl.BlockSpec((blen, _SEQ_BLK), lambda i: (0, i)),
            pl.BlockSpec((vocab, emb_sz), lambda i: (0, 0)),
        ],
        out_specs=pl.BlockSpec((_SEQ_BLK, blen, emb_sz), lambda i: (i, 0, 0)),
        out_shape=jax.ShapeDtypeStruct((seq_len, blen, emb_sz), jnp.float32),
    )(binT, embeddings)

    logitP = pl.pallas_call(
        _logit_body,
        grid=(seq_len // _LG_BLK,),
        in_specs=[
            pl.BlockSpec((blen, _LG_BLK), lambda i: (0, i)),
            pl.BlockSpec((vocab, emb_sz), lambda i: (0, 0)),
        ],
        out_specs=pl.BlockSpec((blen, _LG_BLK // 128, 128),
                               lambda i: (0, i, 0)),
        out_shape=jax.ShapeDtypeStruct((blen, seq_len // 128, 128),
                                       jnp.float32),
    )(binT, embeddings)

    logit = logitP.reshape(blen, seq_len).T.reshape(seq_len, blen, 1)
    return emb, logit


# speedup vs baseline: 1.0115x; 1.0115x over previous
import jax
import jax.numpy as jnp
from jax.experimental import pallas as pl

_SEQ_BLK = 256


def _body(binT_ref, emb_ref, out_ref, logitT_ref):
    ampT = binT_ref[...] * 2.0 - 1.0
    table = emb_ref[...]
    out_ref[...] = ampT.T[:, :, None] * table[None, :, :]
    rowsum = jnp.sum(table, axis=1)
    logitT_ref[...] = ampT * rowsum[:, None]


def kernel(binary_input, embeddings):
    seq_len, blen = binary_input.shape
    vocab, emb_sz = embeddings.shape
    grid = (seq_len // _SEQ_BLK,)
    emb, logitT = pl.pallas_call(
        _body,
        grid=grid,
        in_specs=[
            pl.BlockSpec((blen, _SEQ_BLK), lambda i: (0, i)),
            pl.BlockSpec((vocab, emb_sz), lambda i: (0, 0)),
        ],
        out_specs=(
            pl.BlockSpec((_SEQ_BLK, blen, emb_sz), lambda i: (i, 0, 0)),
            pl.BlockSpec((blen, _SEQ_BLK), lambda i: (0, i)),
        ),
        out_shape=(
            jax.ShapeDtypeStruct((seq_len, blen, emb_sz), jnp.float32),
            jax.ShapeDtypeStruct((blen, seq_len), jnp.float32),
        ),
    )(binary_input.T, embeddings)
    return emb, logitT.T.reshape(seq_len, blen, 1)


# speedup vs baseline: 1.1165x; 1.1038x over previous
import jax
import jax.numpy as jnp
from jax.experimental import pallas as pl

_SEQ_BLK = 512


def _body(binT_ref, emb_ref, out_ref, logitQ_ref):
    ampT = binT_ref[...] * 2.0 - 1.0
    table = emb_ref[...]
    out_ref[...] = ampT.T[:, :, None] * table[None, :, :]
    rowsum = jnp.sum(table, axis=1)
    scaled = ampT * rowsum[:, None]
    for k in range(_SEQ_BLK // 128):
        logitQ_ref[:, k, :, :] = scaled[:, k * 128:(k + 1) * 128].reshape(
            4, 8, 128)


def kernel(binary_input, embeddings):
    seq_len, blen = binary_input.shape
    vocab, emb_sz = embeddings.shape
    grid = (seq_len // _SEQ_BLK,)
    qblk = _SEQ_BLK // 128
    emb, logitQ = pl.pallas_call(
        _body,
        grid=grid,
        in_specs=[
            pl.BlockSpec((blen, _SEQ_BLK), lambda i: (0, i)),
            pl.BlockSpec((vocab, emb_sz), lambda i: (0, 0)),
        ],
        out_specs=(
            pl.BlockSpec((_SEQ_BLK, blen, emb_sz), lambda i: (i, 0, 0)),
            pl.BlockSpec((4, qblk, 8, 128), lambda i: (0, i, 0, 0)),
        ),
        out_shape=(
            jax.ShapeDtypeStruct((seq_len, blen, emb_sz), jnp.float32),
            jax.ShapeDtypeStruct((4, seq_len // 128, 8, 128), jnp.float32),
        ),
    )(binary_input.T, embeddings)
    logit = (logitQ.transpose(0, 2, 1, 3).reshape(blen, seq_len).T
             .reshape(seq_len, blen, 1))
    return emb, logit
